# baseline (speedup 1.0000x reference)
import functools

import numpy as np

import jax
import jax.numpy as jnp
from jax import lax
from jax._src import config as _jax_config
from jax.experimental import pallas as pl
from jax.experimental.pallas import tpu as pltpu
from jax.experimental.pallas import tpu_sc as plsc

_MAX_OBJECTS = 11
_N_TRIPLES = 45
_B = 16384

_OW = 2 * _MAX_OBJECTS
_TW = 2 * 3 * _N_TRIPLES
_SW = 2 * (_MAX_OBJECTS + 2 * _N_TRIPLES)

_NC = 2
_NS = 16
_NW = _NC * _NS
_ROWS_PER_W = _B // _NW
_CHUNK_ROWS = 128
_N_CHUNKS = _ROWS_PER_W // _CHUNK_ROWS
_N_GROUPS = _CHUNK_ROWS // 16


def _sc_body(o_hbm, t_hbm, out_hbm, o_v, t_v, out_v):
    i32 = jnp.int32
    wid = (lax.axis_index("s") * i32(_NC) + lax.axis_index("c")).astype(i32)
    row0 = wid * i32(_ROWS_PER_W)
    lanes = lax.iota(i32, 16)
    zeros = jnp.zeros((16,), i32)

    @pl.loop(np.int32(0), np.int32(_N_CHUNKS))
    def chunk_body(ch):
        base = row0 + ch * i32(_CHUNK_ROWS)
        pltpu.sync_copy(o_hbm.at[pl.ds(base * i32(_OW), _CHUNK_ROWS * _OW)], o_v)
        pltpu.sync_copy(t_hbm.at[pl.ds(base * i32(_TW), _CHUNK_ROWS * _TW)], t_v)

        @pl.loop(np.int32(0), np.int32(_N_GROUPS))
        def group_body(g):
            rows = g * i32(16) + lanes
            dst = rows * i32(_SW)
            ob = rows * i32(_OW)
            for c in range(_MAX_OBJECTS):
                v = plsc.load_gather(o_v, [ob + i32(2 * c)]) + i32(1024)
                plsc.store_scatter(out_v, [dst + i32(2 * c)], v)
                plsc.store_scatter(out_v, [dst + i32(2 * c + 1)], zeros)
            tb = rows * i32(_TW)
            rb = dst + i32(_OW)

            @pl.loop(np.int32(0), np.int32(_N_TRIPLES), unroll=5)
            def trip_body(i):
                six_i = i32(6) * i
                s = plsc.load_gather(t_v, [tb + six_i])
                p = plsc.load_gather(t_v, [tb + (six_i + i32(2))])
                o = plsc.load_gather(t_v, [tb + (six_i + i32(4))])
                even = s * i32(11) + o + i32(1406)
                odd = p + i32(1606)
                d = rb + i32(4) * i
                plsc.store_scatter(out_v, [d], even)
                plsc.store_scatter(out_v, [d + i32(1)], zeros)
                plsc.store_scatter(out_v, [d + i32(2)], odd)
                plsc.store_scatter(out_v, [d + i32(3)], zeros)

        pltpu.sync_copy(out_v, out_hbm.at[pl.ds(base * i32(_SW), _CHUNK_ROWS * _SW)])


@functools.partial(jax.jit, static_argnums=())
def kernel(all_objs, all_triples):
    o32 = lax.bitcast_convert_type(all_objs, jnp.int32).reshape(_B * _OW)
    t32 = lax.bitcast_convert_type(all_triples, jnp.int32).reshape(_B * _TW)

    with _jax_config.enable_x64(False):
        call = pl.kernel(
            _sc_body,
            out_type=jax.ShapeDtypeStruct((_B * _SW,), jnp.int32),
            mesh=plsc.VectorSubcoreMesh(core_axis_name="c", subcore_axis_name="s"),
            scratch_types=[
                pltpu.VMEM((_CHUNK_ROWS * _OW,), jnp.int32),
                pltpu.VMEM((_CHUNK_ROWS * _TW,), jnp.int32),
                pltpu.VMEM((_CHUNK_ROWS * _SW,), jnp.int32),
            ],
            compiler_params=pltpu.CompilerParams(needs_layout_passes=False),
        )
        out32 = call(o32, t32)
    return lax.bitcast_convert_type(out32.reshape(_B, 101, 2), jnp.int64)


# speedup vs baseline: 20.0322x; 20.0322x over previous
import functools

import numpy as np

import jax
import jax.numpy as jnp
from jax import lax
from jax._src import config as _jax_config
from jax.experimental import pallas as pl
from jax.experimental.pallas import tpu as pltpu
from jax.experimental.pallas import tpu_sc as plsc

_MAX_OBJECTS = 11
_N_TRIPLES = 45
_B = 16384
_SEQ = _MAX_OBJECTS + 2 * _N_TRIPLES

_NC = 2
_NS = 16
_NW = _NC * _NS
_BPW = _B // _NW
_CHUNK_B = 128
_N_CHUNKS = _BPW // _CHUNK_B
_N_GROUPS = _CHUNK_B // 16


def _sc_body(o_hbm, t_hbm, out_hbm, o_v, s_v, p_v, ob_v, out_v):
    i32 = jnp.int32
    wid = lax.axis_index("s") * i32(_NC) + lax.axis_index("c")
    b0 = wid * i32(_BPW)

    @pl.loop(np.int32(0), np.int32(_N_CHUNKS))
    def chunk_body(ch):
        base = b0 + ch * i32(_CHUNK_B)
        pltpu.sync_copy(o_hbm.at[:, pl.ds(base, _CHUNK_B)], o_v)
        pltpu.sync_copy(t_hbm.at[0, :, pl.ds(base, _CHUNK_B)], s_v)
        pltpu.sync_copy(t_hbm.at[1, :, pl.ds(base, _CHUNK_B)], p_v)
        pltpu.sync_copy(t_hbm.at[2, :, pl.ds(base, _CHUNK_B)], ob_v)

        @pl.loop(np.int32(0), np.int32(_N_GROUPS))
        def group_body(g):
            l0 = g * i32(16)
            for c in range(_MAX_OBJECTS):
                out_v[c, pl.ds(l0, 16)] = o_v[c, pl.ds(l0, 16)] + i32(1024)

            @pl.loop(np.int32(0), np.int32(_N_TRIPLES), unroll=5)
            def trip_body(i):
                s = s_v[i, pl.ds(l0, 16)]
                p = p_v[i, pl.ds(l0, 16)]
                o = ob_v[i, pl.ds(l0, 16)]
                c = i32(11) + i32(2) * i
                out_v[c, pl.ds(l0, 16)] = s * i32(11) + o + i32(1406)
                out_v[c + i32(1), pl.ds(l0, 16)] = p + i32(1606)

        pltpu.sync_copy(out_v, out_hbm.at[:, pl.ds(base, _CHUNK_B)])


@functools.partial(jax.jit, static_argnums=())
def kernel(all_objs, all_triples):
    o32 = jnp.transpose(all_objs.astype(jnp.int32), (1, 0))
    t32 = jnp.transpose(all_triples.astype(jnp.int32), (2, 1, 0))

    with _jax_config.enable_x64(False):
        call = pl.kernel(
            _sc_body,
            out_type=jax.ShapeDtypeStruct((_SEQ, _B), jnp.int32),
            mesh=plsc.VectorSubcoreMesh(core_axis_name="c", subcore_axis_name="s"),
            scratch_types=[
                pltpu.VMEM((_MAX_OBJECTS, _CHUNK_B), jnp.int32),
                pltpu.VMEM((_N_TRIPLES, _CHUNK_B), jnp.int32),
                pltpu.VMEM((_N_TRIPLES, _CHUNK_B), jnp.int32),
                pltpu.VMEM((_N_TRIPLES, _CHUNK_B), jnp.int32),
                pltpu.VMEM((_SEQ, _CHUNK_B), jnp.int32),
            ],
            compiler_params=pltpu.CompilerParams(
                needs_layout_passes=False,
                use_tc_tiling_on_sc=True,
                disable_bounds_checks=True,
            ),
        )
        out32 = call(o32, t32)
    return jnp.transpose(out32, (1, 0)).astype(jnp.int64)


# speedup vs baseline: 20.7324x; 1.0350x over previous
import functools

import numpy as np

import jax
import jax.numpy as jnp
from jax import lax
from jax._src import config as _jax_config
from jax.experimental import pallas as pl
from jax.experimental.pallas import tpu as pltpu
from jax.experimental.pallas import tpu_sc as plsc

_MAX_OBJECTS = 11
_N_TRIPLES = 45
_B = 16384
_SEQ = _MAX_OBJECTS + 2 * _N_TRIPLES

_NC = 2
_NS = 16
_NW = _NC * _NS
_BPW = _B // _NW
_CHUNK_B = 128
_N_CHUNKS = _BPW // _CHUNK_B
_N_GROUPS = _CHUNK_B // 16


def _sc_body(o_hbm, t_hbm, out_hbm, o_v, s_v, p_v, ob_v, out_v):
    i32 = jnp.int32
    u32 = jnp.uint32
    wid = lax.axis_index("s") * i32(_NC) + lax.axis_index("c")
    b0 = wid * i32(_BPW)

    @pl.loop(np.int32(0), np.int32(_N_CHUNKS))
    def chunk_body(ch):
        base = b0 + ch * i32(_CHUNK_B)
        pltpu.sync_copy(o_hbm.at[:, pl.ds(base, _CHUNK_B)], o_v)
        pltpu.sync_copy(t_hbm.at[0, :, pl.ds(base, _CHUNK_B)], s_v)
        pltpu.sync_copy(t_hbm.at[1, :, pl.ds(base, _CHUNK_B)], p_v)
        pltpu.sync_copy(t_hbm.at[2, :, pl.ds(base, _CHUNK_B)], ob_v)

        @pl.loop(np.int32(0), np.int32(_N_GROUPS))
        def group_body(g):
            l0 = g * i32(16)
            for c in range(_MAX_OBJECTS):
                out_v[c, pl.ds(l0, 16)] = o_v[c, pl.ds(l0, 16)] + u32(1024)

            @pl.loop(np.int32(0), np.int32(_N_TRIPLES), unroll=5)
            def trip_body(i):
                s = s_v[i, pl.ds(l0, 16)]
                p = p_v[i, pl.ds(l0, 16)]
                o = ob_v[i, pl.ds(l0, 16)]
                c = i32(11) + i32(2) * i
                out_v[c, pl.ds(l0, 16)] = s * u32(11) + o + u32(1406)
                out_v[c + i32(1), pl.ds(l0, 16)] = p + u32(1606)

        pltpu.sync_copy(out_v, out_hbm.at[:, pl.ds(base, _CHUNK_B)])


@functools.partial(jax.jit, static_argnums=())
def kernel(all_objs, all_triples):
    o32 = jnp.transpose(all_objs.astype(jnp.uint32), (1, 0))
    t32 = jnp.transpose(all_triples.astype(jnp.uint32), (2, 1, 0))

    with _jax_config.enable_x64(False):
        call = pl.kernel(
            _sc_body,
            out_type=jax.ShapeDtypeStruct((_SEQ, _B), jnp.uint32),
            mesh=plsc.VectorSubcoreMesh(core_axis_name="c", subcore_axis_name="s"),
            scratch_types=[
                pltpu.VMEM((_MAX_OBJECTS, _CHUNK_B), jnp.uint32),
                pltpu.VMEM((_N_TRIPLES, _CHUNK_B), jnp.uint32),
                pltpu.VMEM((_N_TRIPLES, _CHUNK_B), jnp.uint32),
                pltpu.VMEM((_N_TRIPLES, _CHUNK_B), jnp.uint32),
                pltpu.VMEM((_SEQ, _CHUNK_B), jnp.uint32),
            ],
            compiler_params=pltpu.CompilerParams(
                needs_layout_passes=False,
                use_tc_tiling_on_sc=True,
                disable_bounds_checks=True,
            ),
        )
        out32 = call(o32, t32)
    return jnp.transpose(out32, (1, 0)).astype(jnp.int64)


# speedup vs baseline: 21.8034x; 1.0517x over previous
import functools

import numpy as np

import jax
import jax.numpy as jnp
from jax import lax
from jax._src import config as _jax_config
from jax.experimental import pallas as pl
from jax.experimental.pallas import tpu as pltpu
from jax.experimental.pallas import tpu_sc as plsc

_MAX_OBJECTS = 11
_N_TRIPLES = 45
_B = 16384
_SEQ = _MAX_OBJECTS + 2 * _N_TRIPLES

_NC = 2
_NS = 16
_NW = _NC * _NS
_BPW = _B // _NW
_CHUNK_B = 128
_N_CHUNKS = _BPW // _CHUNK_B
_N_GROUPS = _CHUNK_B // 16


def _sc_body(o_hbm, t_hbm, out_hbm,
             o_v0, s_v0, p_v0, ob_v0, out_v0,
             o_v1, s_v1, p_v1, ob_v1, out_v1,
             in_sem0, in_sem1, out_sem0, out_sem1):
    i32 = jnp.int32
    u32 = jnp.uint32
    wid = lax.axis_index("s") * i32(_NC) + lax.axis_index("c")
    b0 = wid * i32(_BPW)

    bufs = ((o_v0, s_v0, p_v0, ob_v0, out_v0, in_sem0, out_sem0),
            (o_v1, s_v1, p_v1, ob_v1, out_v1, in_sem1, out_sem1))

    def start_in(ch, j):
        o_v, s_v, p_v, ob_v, _, in_sem, _ = bufs[j]
        base = b0 + i32(ch * _CHUNK_B)
        return (
            pltpu.async_copy(o_hbm.at[:, pl.ds(base, _CHUNK_B)], o_v, in_sem),
            pltpu.async_copy(t_hbm.at[0, :, pl.ds(base, _CHUNK_B)], s_v, in_sem),
            pltpu.async_copy(t_hbm.at[1, :, pl.ds(base, _CHUNK_B)], p_v, in_sem),
            pltpu.async_copy(t_hbm.at[2, :, pl.ds(base, _CHUNK_B)], ob_v, in_sem),
        )

    def compute(j):
        o_v, s_v, p_v, ob_v, out_v, _, _ = bufs[j]

        @pl.loop(np.int32(0), np.int32(_N_GROUPS))
        def group_body(g):
            l0 = g * i32(16)
            for c in range(_MAX_OBJECTS):
                out_v[c, pl.ds(l0, 16)] = o_v[c, pl.ds(l0, 16)] + u32(1024)

            @pl.loop(np.int32(0), np.int32(_N_TRIPLES), unroll=5)
            def trip_body(i):
                s = s_v[i, pl.ds(l0, 16)]
                p = p_v[i, pl.ds(l0, 16)]
                o = ob_v[i, pl.ds(l0, 16)]
                c = i32(11) + i32(2) * i
                out_v[c, pl.ds(l0, 16)] = s * u32(11) + o + u32(1406)
                out_v[c + i32(1), pl.ds(l0, 16)] = p + u32(1606)

    cps = start_in(0, 0)
    out_cp = [None, None]
    for ch in range(_N_CHUNKS):
        j = ch % 2
        nxt = start_in(ch + 1, 1 - j) if ch + 1 < _N_CHUNKS else ()
        for cp in cps:
            cp.wait()
        cps = nxt
        if out_cp[j] is not None:
            out_cp[j].wait()
        compute(j)
        out_v, out_sem = bufs[j][4], bufs[j][6]
        base = b0 + i32(ch * _CHUNK_B)
        out_cp[j] = pltpu.async_copy(
            out_v, out_hbm.at[:, pl.ds(base, _CHUNK_B)], out_sem)
    for cp in out_cp:
        if cp is not None:
            cp.wait()


@functools.partial(jax.jit, static_argnums=())
def kernel(all_objs, all_triples):
    o32 = jnp.transpose(all_objs.astype(jnp.uint32), (1, 0))
    t32 = jnp.transpose(all_triples.astype(jnp.uint32), (2, 1, 0))

    with _jax_config.enable_x64(False):
        call = pl.kernel(
            _sc_body,
            out_type=jax.ShapeDtypeStruct((_SEQ, _B), jnp.uint32),
            mesh=plsc.VectorSubcoreMesh(core_axis_name="c", subcore_axis_name="s"),
            scratch_types=(
                [pltpu.VMEM((_MAX_OBJECTS, _CHUNK_B), jnp.uint32),
                 pltpu.VMEM((_N_TRIPLES, _CHUNK_B), jnp.uint32),
                 pltpu.VMEM((_N_TRIPLES, _CHUNK_B), jnp.uint32),
                 pltpu.VMEM((_N_TRIPLES, _CHUNK_B), jnp.uint32),
                 pltpu.VMEM((_SEQ, _CHUNK_B), jnp.uint32)] * 2
                + [pltpu.SemaphoreType.DMA] * 4
            ),
            compiler_params=pltpu.CompilerParams(
                needs_layout_passes=False,
                use_tc_tiling_on_sc=True,
                disable_bounds_checks=True,
            ),
        )
        out32 = call(o32, t32)
    return jnp.transpose(out32, (1, 0)).astype(jnp.int64)
